# baseline (speedup 1.0000x reference)
import jax
import jax.numpy as jnp
from jax.experimental import pallas as pl
from jax.experimental.pallas import tpu as pltpu

_BM = 400


def _fused_kernel(a_hbm, x_ref, w_ref, b_ref, o_ref, h_ref, abuf, sem):
    i = pl.program_id(0)
    nb = pl.num_programs(0)

    @pl.when(i == 0)
    def _():
        pltpu.make_async_copy(a_hbm.at[pl.ds(0, _BM), :], abuf.at[0],
                              sem.at[0]).start()
        h_ref[...] = jnp.dot(x_ref[...], w_ref[...],
                             preferred_element_type=jnp.float32)

    @pl.when(i + 1 < nb)
    def _():
        slot = (i + 1) % 2
        pltpu.make_async_copy(a_hbm.at[pl.ds((i + 1) * _BM, _BM), :],
                              abuf.at[slot], sem.at[slot]).start()

    slot = i % 2
    pltpu.make_async_copy(a_hbm.at[pl.ds(i * _BM, _BM), :], abuf.at[slot],
                          sem.at[slot]).wait()
    acc = jnp.dot(abuf[slot], h_ref[...], preferred_element_type=jnp.float32)
    o_ref[...] = jnp.maximum(acc + b_ref[...], 0.0)


def kernel(A, x, W, b):
    N, D = x.shape
    return pl.pallas_call(
        _fused_kernel,
        grid=(N // _BM,),
        in_specs=[
            pl.BlockSpec(memory_space=pltpu.MemorySpace.HBM),
            pl.BlockSpec((N, D), lambda i: (0, 0)),
            pl.BlockSpec((D, D), lambda i: (0, 0)),
            pl.BlockSpec((1, D), lambda i: (0, 0)),
        ],
        out_specs=pl.BlockSpec((_BM, D), lambda i: (i, 0)),
        out_shape=jax.ShapeDtypeStruct((N, D), jnp.float32),
        scratch_shapes=[
            pltpu.VMEM((N, D), jnp.float32),
            pltpu.VMEM((2, _BM, N), jnp.float32),
            pltpu.SemaphoreType.DMA((2,)),
        ],
    )(A, x, W, b.reshape(1, D))


# speedup vs baseline: 1.0010x; 1.0010x over previous
import jax
import jax.numpy as jnp
from jax.experimental import pallas as pl
from jax.experimental.pallas import tpu as pltpu

_BM = 400


_NCHUNK = 5
_CH = _BM // _NCHUNK


def _tile_copies(a_hbm, abuf, sem, tile, slot):
    for c in range(_NCHUNK):
        yield pltpu.make_async_copy(
            a_hbm.at[pl.ds(tile * _BM + c * _CH, _CH), :],
            abuf.at[slot, pl.ds(c * _CH, _CH), :],
            sem.at[slot, c])


def _fused_kernel(a_hbm, x_ref, w_ref, b_ref, o_ref, h_ref, abuf, sem):
    i = pl.program_id(0)
    nb = pl.num_programs(0)

    @pl.when(i == 0)
    def _():
        for cp in _tile_copies(a_hbm, abuf, sem, 0, 0):
            cp.start()
        h_ref[...] = jnp.dot(x_ref[...], w_ref[...],
                             preferred_element_type=jnp.float32)

    @pl.when(i + 1 < nb)
    def _():
        slot = (i + 1) % 2
        for cp in _tile_copies(a_hbm, abuf, sem, i + 1, slot):
            cp.start()

    slot = i % 2
    for cp in _tile_copies(a_hbm, abuf, sem, i, slot):
        cp.wait()
    acc = jnp.dot(abuf[slot], h_ref[...], preferred_element_type=jnp.float32)
    o_ref[...] = jnp.maximum(acc + b_ref[...], 0.0)


def kernel(A, x, W, b):
    N, D = x.shape
    return pl.pallas_call(
        _fused_kernel,
        grid=(N // _BM,),
        in_specs=[
            pl.BlockSpec(memory_space=pltpu.MemorySpace.HBM),
            pl.BlockSpec((N, D), lambda i: (0, 0)),
            pl.BlockSpec((D, D), lambda i: (0, 0)),
            pl.BlockSpec((1, D), lambda i: (0, 0)),
        ],
        out_specs=pl.BlockSpec((_BM, D), lambda i: (i, 0)),
        out_shape=jax.ShapeDtypeStruct((N, D), jnp.float32),
        scratch_shapes=[
            pltpu.VMEM((N, D), jnp.float32),
            pltpu.VMEM((2, _BM, N), jnp.float32),
            pltpu.SemaphoreType.DMA((2, _NCHUNK)),
        ],
    )(A, x, W, b.reshape(1, D))
